# Initial kernel scaffold; baseline (speedup 1.0000x reference)
import jax
import jax.numpy as jnp
from jax.experimental import pallas as pl


def kernel(input_image_re, input_image_im, input_mask, output_re, output_im, w1_re, w1_im, w2_re, w2_im, L1_re, L1_im, L2_re, L2_im):
    raise NotImplementedError("write your pallas kernel here")


# speedup vs baseline: 2.6329x; 2.6329x over previous
import functools
import numpy as np
import jax
import jax.numpy as jnp
from jax import lax
from jax.experimental import pallas as pl
from jax.experimental.pallas import tpu as pltpu
from jax.experimental.pallas import tpu_sc as plsc

H = 384
W = 384
IN_F = 64
OUT_F = 64
RING_W = 4
MAX_NEIGH = 128
N = H * W
NR = 68
NRP = 72
BLK = 256
NW = 32
CH = 128


def _precompute():
    c0 = np.arange(H, dtype=np.float64) - H / 2.0
    c1 = np.arange(W, dtype=np.float64) - W / 2.0
    X, Y = np.meshgrid(c0, c1, indexing='ij')
    r = np.sqrt(X * X + Y * Y).reshape(-1)
    phi = np.arctan2(X, Y).reshape(-1)
    ring = (r / RING_W).astype(np.int64)
    counts = np.bincount(ring, minlength=NR)
    blocks_per_ring = (counts + BLK - 1) // BLK
    offs = np.zeros(NR + 1, np.int64)
    offs[1:] = np.cumsum(blocks_per_ring * BLK)
    P = int(offs[-1])
    gidx = np.zeros(P, np.int32)
    st = np.zeros((P, 4), np.float32)
    rob, fob = [], []
    nbr_gidx = np.zeros(NR * MAX_NEIGH, np.int32)
    nbr_w = np.zeros((NRP, NR * MAX_NEIGH), np.float32)
    nbr_k = np.ones((NRP, 1), np.float32)
    for rr in range(NR):
        idx = np.nonzero(ring == rr)[0]
        n = idx.size
        o = int(offs[rr])
        gidx[o:o + n] = idx
        st[o:o + n, 0] = r[idx]
        st[o:o + n, 1] = phi[idx]
        st[o:o + n, 2] = 1.0
        for b in range(int(blocks_per_ring[rr])):
            rob.append(rr)
            fob.append(1 if b == 0 else 0)
        if n > MAX_NEIGH:
            rng = np.random.default_rng(rr)
            sel = rng.choice(n - 1, size=MAX_NEIGH, replace=False)
            k = MAX_NEIGH
            nbr_gidx[rr * MAX_NEIGH: rr * MAX_NEIGH + k] = idx[sel]
        else:
            k = n
            nbr_gidx[rr * MAX_NEIGH: rr * MAX_NEIGH + k] = idx
        nbr_w[rr, rr * MAX_NEIGH: rr * MAX_NEIGH + k] = 1.0
        nbr_k[rr, 0] = float(k)
    rob = np.array(rob, np.int32)
    fob = np.array(fob, np.int32)
    inv = np.zeros(N, np.int32)
    valid = st[:, 2] > 0
    inv[gidx[valid]] = np.nonzero(valid)[0].astype(np.int32)
    M_raw = P + NR * MAX_NEIGH
    M = ((M_raw + NW * CH - 1) // (NW * CH)) * (NW * CH)
    idx_all = np.zeros(M, np.int32)
    idx_all[:P] = gidx
    idx_all[P:M_raw] = nbr_gidx
    return dict(P=P, M=M, NB=len(rob), idx_all=idx_all, st=st, rob=rob,
                fob=fob, nbr_w=nbr_w, nbr_k=nbr_k, inv=inv)


_PC = _precompute()


def _sc_gather(tables, idx, n_out):
    nch = n_out // (NW * CH)
    bpw = n_out // NW
    mesh = plsc.VectorSubcoreMesh(core_axis_name="c", subcore_axis_name="s",
                                  num_cores=2, num_subcores=16)
    dims = [t.shape[1] for t in tables]
    out_type = [jax.ShapeDtypeStruct((n_out, d), jnp.float32) for d in dims]
    scratch = [pltpu.VMEM((CH,), jnp.int32)]
    scratch += [pltpu.VMEM((CH, d), jnp.float32) for d in dims]
    scratch += [pltpu.SemaphoreType.DMA]

    @functools.partial(pl.kernel, mesh=mesh, out_type=out_type,
                       scratch_types=scratch,
                       compiler_params=pltpu.CompilerParams(
                           use_tc_tiling_on_sc=False))
    def k(*refs):
        nt = len(dims)
        tabs = refs[:nt]
        idx_hbm = refs[nt]
        outs = refs[nt + 1: 2 * nt + 1]
        idx_v = refs[2 * nt + 1]
        bufs = refs[2 * nt + 2: 3 * nt + 2]
        sem = refs[3 * nt + 2]
        wid = lax.axis_index("s") * 2 + lax.axis_index("c")
        base = wid * bpw

        def body(i, carry):
            off = base + i * CH
            pltpu.sync_copy(idx_hbm.at[pl.ds(off, CH)], idx_v)
            cps = [pltpu.async_copy(t.at[idx_v], b, sem)
                   for t, b in zip(tabs, bufs)]
            for c in cps:
                c.wait()
            for b, o in zip(bufs, outs):
                pltpu.sync_copy(b, o.at[pl.ds(off, CH)])
            return carry

        lax.fori_loop(0, nch, body, 0)

    return k(*tables, idx)


def _dot(a, b):
    return jnp.dot(a, b, preferred_element_type=jnp.float32,
                   precision=lax.Precision.DEFAULT)


def _dot_hi(a, b):
    return jnp.dot(a, b, preferred_element_type=jnp.float32,
                   precision=lax.Precision.HIGHEST)


def _s_kernel(nbr_re, nbr_im, nbr_m, wfull, kcol, w1_re, w1_im):
    def body(nre, nim, nm, wf, kc, w1r, w1i, sre, sim):
        u = (nm[:, 0:1] == 0.0).astype(jnp.float32)
        ur = nre[...] * u
        ui = nim[...] * u
        nw1r = _dot(ur, w1r[...]) - _dot(ui, w1i[...])
        nw1i = _dot(ur, w1i[...]) + _dot(ui, w1r[...])
        sre[...] = _dot_hi(wf[...], nw1r) / kc[...]
        sim[...] = _dot_hi(wf[...], nw1i) / kc[...]

    return pl.pallas_call(
        body,
        out_shape=[jax.ShapeDtypeStruct((NRP, OUT_F), jnp.float32)] * 2,
    )(nbr_re, nbr_im, nbr_m, wfull, kcol, w1_re, w1_im)


def _compute_o(are, aim, m, st, sre, sim, w1r, w1i, w2r, w2i, rid):
    valid = st[:, 2:3]
    mm = m[:, 0:1]
    uncor = (mm == 0.0).astype(jnp.float32) * valid
    cor = (mm == 1.0).astype(jnp.float32) * valid
    un_re = are * uncor
    un_im = aim * uncor
    co_re = are * cor
    co_im = aim * cor
    s_re = sre[pl.ds(rid, 1), :]
    s_im = sim[pl.ds(rid, 1), :]
    mul_re = _dot(un_re, w1r) - _dot(un_im, w1i)
    mul_im = _dot(un_re, w1i) + _dot(un_im, w1r)
    c_re = _dot(co_re, w2r) - _dot(co_im, w2i)
    c_im = _dot(co_re, w2i) + _dot(co_im, w2r)
    uo_re = 0.5 * (mul_re + s_re) * uncor
    uo_im = 0.5 * (mul_im + s_im) * uncor
    o_re = uo_re + (c_re - s_re) * cor
    o_im = uo_im + (c_im - s_im) * cor
    return o_re, o_im


def _pass1(scalars, arrays):
    rob, fob = scalars
    NB = _PC["NB"]

    def body(rob_ref, fob_ref, are, aim, m, st, sre, sim,
             w1r, w1i, w2r, w2i, l1r, l1i, l2r, l2i, g_ref):
        i = pl.program_id(0)
        rid = rob_ref[i]
        o_re, o_im = _compute_o(are[...], aim[...], m[...], st[...],
                                sre, sim, w1r[...], w1i[...],
                                w2r[...], w2i[...], rid)
        ro = st[:, 0:2]
        comb_re = jnp.concatenate([o_re, ro], axis=1)
        comb_im = jnp.concatenate([o_im, jnp.zeros_like(ro)], axis=1)
        m1r = _dot(comb_re, l1r[...]) - _dot(comb_im, l1i[...])
        m1i = _dot(comb_re, l1i[...]) + _dot(comb_im, l1r[...])
        m2r = _dot(comb_re, l2r[...]) - _dot(comb_im, l2i[...])
        m2i = _dot(comb_re, l2i[...]) + _dot(comb_im, l2r[...])
        m1 = jnp.concatenate([m1r, m1i], axis=1)
        m2 = jnp.concatenate([m2r, m2i], axis=1)
        gpart = lax.dot_general(m1, m2, (((0,), (0,)), ((), ())),
                                preferred_element_type=jnp.float32,
                                precision=lax.Precision.DEFAULT)

        @pl.when(fob_ref[i] == 1)
        def _():
            g_ref[...] = gpart[None]

        @pl.when(fob_ref[i] == 0)
        def _():
            g_ref[...] += gpart[None]

    full = lambda shape: pl.BlockSpec(shape, lambda i, *_: (0,) * len(shape))
    rowblk = lambda d: pl.BlockSpec((BLK, d), lambda i, *_: (i, 0))
    grid_spec = pltpu.PrefetchScalarGridSpec(
        num_scalar_prefetch=2,
        grid=(NB,),
        in_specs=[
            rowblk(64), rowblk(64), rowblk(16), rowblk(4),
            full((NRP, OUT_F)), full((NRP, OUT_F)),
            full((IN_F, OUT_F)), full((IN_F, OUT_F)),
            full((IN_F, OUT_F)), full((IN_F, OUT_F)),
            full((IN_F + 2, OUT_F)), full((IN_F + 2, OUT_F)),
            full((IN_F + 2, OUT_F)), full((IN_F + 2, OUT_F)),
        ],
        out_specs=pl.BlockSpec((1, 2 * OUT_F, 2 * OUT_F),
                               lambda i, rob_ref, fob_ref: (rob_ref[i], 0, 0)),
    )
    return pl.pallas_call(
        body,
        grid_spec=grid_spec,
        out_shape=jax.ShapeDtypeStruct((NRP, 2 * OUT_F, 2 * OUT_F),
                                       jnp.float32),
    )(rob, fob, *arrays)


def _pass2(scalars, arrays):
    rob, fob = scalars
    NB = _PC["NB"]
    P = _PC["P"]

    def body(rob_ref, fob_ref, are, aim, m, st, sre, sim,
             w1r, w1i, w2r, w2i, g_ref, fre, fim):
        i = pl.program_id(0)
        rid = rob_ref[i]
        o_re, o_im = _compute_o(are[...], aim[...], m[...], st[...],
                                sre, sim, w1r[...], w1i[...],
                                w2r[...], w2i[...], rid)
        g = g_ref[0]
        mul_re = jax.nn.sigmoid(g[0:OUT_F, 0:OUT_F]
                                - g[OUT_F:2 * OUT_F, OUT_F:2 * OUT_F])
        mul_im = jax.nn.sigmoid(g[0:OUT_F, OUT_F:2 * OUT_F]
                                + g[OUT_F:2 * OUT_F, 0:OUT_F])
        fre[...] = _dot(o_re, mul_re) - _dot(o_im, mul_im)
        fim[...] = _dot(o_re, mul_im) + _dot(o_im, mul_re)

    full = lambda shape: pl.BlockSpec(shape, lambda i, *_: (0,) * len(shape))
    rowblk = lambda d: pl.BlockSpec((BLK, d), lambda i, *_: (i, 0))
    grid_spec = pltpu.PrefetchScalarGridSpec(
        num_scalar_prefetch=2,
        grid=(NB,),
        in_specs=[
            rowblk(64), rowblk(64), rowblk(16), rowblk(4),
            full((NRP, OUT_F)), full((NRP, OUT_F)),
            full((IN_F, OUT_F)), full((IN_F, OUT_F)),
            full((IN_F, OUT_F)), full((IN_F, OUT_F)),
            pl.BlockSpec((1, 2 * OUT_F, 2 * OUT_F),
                         lambda i, rob_ref, fob_ref: (rob_ref[i], 0, 0)),
        ],
        out_specs=[
            pl.BlockSpec((BLK, OUT_F), lambda i, *_: (i, 0)),
            pl.BlockSpec((BLK, OUT_F), lambda i, *_: (i, 0)),
        ],
    )
    return pl.pallas_call(
        body,
        grid_spec=grid_spec,
        out_shape=[jax.ShapeDtypeStruct((P, OUT_F), jnp.float32)] * 2,
    )(rob, fob, *arrays)


def kernel(input_image_re, input_image_im, input_mask, output_re, output_im,
           w1_re, w1_im, w2_re, w2_im, L1_re, L1_im, L2_re, L2_im):
    pc = _PC
    P, M = pc["P"], pc["M"]
    a_re = input_image_re.reshape(N, IN_F)
    a_im = input_image_im.reshape(N, IN_F)
    mask_t = jnp.tile(input_mask.reshape(N, 1).astype(jnp.float32), (1, 16))
    idx_all = jnp.asarray(pc["idx_all"])
    g_re, g_im, g_m = _sc_gather([a_re, a_im, mask_t], idx_all, M)

    nbr_re = lax.slice(g_re, (P, 0), (P + NR * MAX_NEIGH, IN_F))
    nbr_im = lax.slice(g_im, (P, 0), (P + NR * MAX_NEIGH, IN_F))
    nbr_m = lax.slice(g_m, (P, 0), (P + NR * MAX_NEIGH, 16))
    wfull = jnp.asarray(pc["nbr_w"])
    kcol = jnp.asarray(pc["nbr_k"])
    s_re, s_im = _s_kernel(nbr_re, nbr_im, nbr_m, wfull, kcol, w1_re, w1_im)

    rob = jnp.asarray(pc["rob"])
    fob = jnp.asarray(pc["fob"])
    st = jnp.asarray(pc["st"])
    g = _pass1((rob, fob),
               [g_re, g_im, g_m, st, s_re, s_im,
                w1_re, w1_im, w2_re, w2_im, L1_re, L1_im, L2_re, L2_im])
    f_re, f_im = _pass2((rob, fob),
                        [g_re, g_im, g_m, st, s_re, s_im,
                         w1_re, w1_im, w2_re, w2_im, g])

    inv = jnp.asarray(pc["inv"])
    out_re, out_im = _sc_gather([f_re, f_im], inv, N)
    out_re = out_re.reshape(H, W, OUT_F)
    out_im = out_im.reshape(H, W, OUT_F)
    return out_re + 1j * out_im


# speedup vs baseline: 2.6348x; 1.0007x over previous
import functools
import numpy as np
import jax
import jax.numpy as jnp
from jax import lax
from jax.experimental import pallas as pl
from jax.experimental.pallas import tpu as pltpu
from jax.experimental.pallas import tpu_sc as plsc

H = 384
W = 384
IN_F = 64
OUT_F = 64
RING_W = 4
MAX_NEIGH = 128
N = H * W
NR = 68
NRP = 72
BLK = 256
NW = 32
CH = 128


def _precompute():
    c0 = np.arange(H, dtype=np.float64) - H / 2.0
    c1 = np.arange(W, dtype=np.float64) - W / 2.0
    X, Y = np.meshgrid(c0, c1, indexing='ij')
    r = np.sqrt(X * X + Y * Y).reshape(-1)
    phi = np.arctan2(X, Y).reshape(-1)
    ring = (r / RING_W).astype(np.int64)
    counts = np.bincount(ring, minlength=NR)
    blocks_per_ring = (counts + BLK - 1) // BLK
    offs = np.zeros(NR + 1, np.int64)
    offs[1:] = np.cumsum(blocks_per_ring * BLK)
    P = int(offs[-1])
    gidx = np.zeros(P, np.int32)
    st = np.zeros((P, 4), np.float32)
    rob, fob = [], []
    nbr_gidx = np.zeros(NR * MAX_NEIGH, np.int32)
    nbr_w = np.zeros((NRP, NR * MAX_NEIGH), np.float32)
    nbr_k = np.ones((NRP, 1), np.float32)
    for rr in range(NR):
        idx = np.nonzero(ring == rr)[0]
        n = idx.size
        o = int(offs[rr])
        gidx[o:o + n] = idx
        st[o:o + n, 0] = r[idx]
        st[o:o + n, 1] = phi[idx]
        st[o:o + n, 2] = 1.0
        for b in range(int(blocks_per_ring[rr])):
            rob.append(rr)
            fob.append(1 if b == 0 else 0)
        if n > MAX_NEIGH:
            rng = np.random.default_rng(rr)
            sel = rng.choice(n - 1, size=MAX_NEIGH, replace=False)
            k = MAX_NEIGH
            nbr_gidx[rr * MAX_NEIGH: rr * MAX_NEIGH + k] = idx[sel]
        else:
            k = n
            nbr_gidx[rr * MAX_NEIGH: rr * MAX_NEIGH + k] = idx
        nbr_w[rr, rr * MAX_NEIGH: rr * MAX_NEIGH + k] = 1.0
        nbr_k[rr, 0] = float(k)
    rob = np.array(rob, np.int32)
    fob = np.array(fob, np.int32)
    inv = np.zeros(N, np.int32)
    valid = st[:, 2] > 0
    inv[gidx[valid]] = np.nonzero(valid)[0].astype(np.int32)
    M_raw = P + NR * MAX_NEIGH
    M = ((M_raw + NW * CH - 1) // (NW * CH)) * (NW * CH)
    idx_all = np.zeros(M, np.int32)
    idx_all[:P] = gidx
    idx_all[P:M_raw] = nbr_gidx
    return dict(P=P, M=M, NB=len(rob), idx_all=idx_all, st=st, rob=rob,
                fob=fob, nbr_w=nbr_w, nbr_k=nbr_k, inv=inv)


_PC = _precompute()


def _sc_gather(tables, idx, n_out):
    nch = n_out // (NW * CH)
    bpw = n_out // NW
    mesh = plsc.VectorSubcoreMesh(core_axis_name="c", subcore_axis_name="s",
                                  num_cores=2, num_subcores=16)
    dims = [t.shape[1] for t in tables]
    out_type = [jax.ShapeDtypeStruct((n_out, d), jnp.float32) for d in dims]
    scratch = [pltpu.VMEM((CH,), jnp.int32)]
    scratch += [pltpu.VMEM((CH, d), jnp.float32) for d in dims]
    scratch += [pltpu.SemaphoreType.DMA]

    @functools.partial(pl.kernel, mesh=mesh, out_type=out_type,
                       scratch_types=scratch,
                       compiler_params=pltpu.CompilerParams(
                           use_tc_tiling_on_sc=False))
    def k(*refs):
        nt = len(dims)
        tabs = refs[:nt]
        idx_hbm = refs[nt]
        outs = refs[nt + 1: 2 * nt + 1]
        idx_v = refs[2 * nt + 1]
        bufs = refs[2 * nt + 2: 3 * nt + 2]
        sem = refs[3 * nt + 2]
        wid = lax.axis_index("s") * 2 + lax.axis_index("c")
        base = wid * bpw

        def body(i, carry):
            off = base + i * CH
            pltpu.sync_copy(idx_hbm.at[pl.ds(off, CH)], idx_v)
            cps = [pltpu.async_copy(t.at[idx_v], b, sem)
                   for t, b in zip(tabs, bufs)]
            for c in cps:
                c.wait()
            for b, o in zip(bufs, outs):
                pltpu.sync_copy(b, o.at[pl.ds(off, CH)])
            return carry

        lax.fori_loop(0, nch, body, 0)

    return k(*tables, idx)


def _dot(a, b):
    return jnp.dot(a, b, preferred_element_type=jnp.float32,
                   precision=lax.Precision.DEFAULT)


def _dot_hi(a, b):
    return jnp.dot(a, b, preferred_element_type=jnp.float32,
                   precision=lax.Precision.HIGHEST)


def _s_kernel(nbr_re, nbr_im, nbr_m, wfull, kcol, w1_re, w1_im):
    def body(nre, nim, nm, wf, kc, w1r, w1i, sre, sim):
        u = (nm[:, 0:1] == 0.0).astype(jnp.float32)
        ur = nre[...] * u
        ui = nim[...] * u
        nw1r = _dot(ur, w1r[...]) - _dot(ui, w1i[...])
        nw1i = _dot(ur, w1i[...]) + _dot(ui, w1r[...])
        sre[...] = _dot_hi(wf[...], nw1r) / kc[...]
        sim[...] = _dot_hi(wf[...], nw1i) / kc[...]

    return pl.pallas_call(
        body,
        out_shape=[jax.ShapeDtypeStruct((NRP, OUT_F), jnp.float32)] * 2,
    )(nbr_re, nbr_im, nbr_m, wfull, kcol, w1_re, w1_im)


def _compute_o(are, aim, m, st, sre, sim, w1r, w1i, w2r, w2i, rid):
    valid = st[:, 2:3]
    mm = m[:, 0:1]
    uncor = (mm == 0.0).astype(jnp.float32) * valid
    cor = (mm == 1.0).astype(jnp.float32) * valid
    un_re = are * uncor
    un_im = aim * uncor
    co_re = are * cor
    co_im = aim * cor
    s_re = sre[pl.ds(rid, 1), :]
    s_im = sim[pl.ds(rid, 1), :]
    mul_re = _dot(un_re, w1r) - _dot(un_im, w1i)
    mul_im = _dot(un_re, w1i) + _dot(un_im, w1r)
    c_re = _dot(co_re, w2r) - _dot(co_im, w2i)
    c_im = _dot(co_re, w2i) + _dot(co_im, w2r)
    uo_re = 0.5 * (mul_re + s_re) * uncor
    uo_im = 0.5 * (mul_im + s_im) * uncor
    o_re = uo_re + (c_re - s_re) * cor
    o_im = uo_im + (c_im - s_im) * cor
    return o_re, o_im


def _pass1(scalars, arrays):
    rob, fob = scalars
    NB = _PC["NB"]

    def body(rob_ref, fob_ref, are, aim, m, st, sre, sim,
             w1r, w1i, w2r, w2i, l1r, l1i, l2r, l2i, g_ref):
        i = pl.program_id(0)
        rid = rob_ref[i]
        o_re, o_im = _compute_o(are[...], aim[...], m[...], st[...],
                                sre, sim, w1r[...], w1i[...],
                                w2r[...], w2i[...], rid)
        ro = st[:, 0:2]
        comb_re = jnp.concatenate([o_re, ro], axis=1)
        comb_im = jnp.concatenate([o_im, jnp.zeros_like(ro)], axis=1)
        m1r = _dot(comb_re, l1r[...]) - _dot(comb_im, l1i[...])
        m1i = _dot(comb_re, l1i[...]) + _dot(comb_im, l1r[...])
        m2r = _dot(comb_re, l2r[...]) - _dot(comb_im, l2i[...])
        m2i = _dot(comb_re, l2i[...]) + _dot(comb_im, l2r[...])
        m1 = jnp.concatenate([m1r, m1i], axis=1)
        m2 = jnp.concatenate([m2r, m2i], axis=1)
        gpart = lax.dot_general(m1, m2, (((0,), (0,)), ((), ())),
                                preferred_element_type=jnp.float32,
                                precision=lax.Precision.DEFAULT)

        @pl.when(fob_ref[i] == 1)
        def _():
            g_ref[...] = gpart[None]

        @pl.when(fob_ref[i] == 0)
        def _():
            g_ref[...] += gpart[None]

    full = lambda shape: pl.BlockSpec(shape, lambda i, *_: (0,) * len(shape))
    rowblk = lambda d: pl.BlockSpec((BLK, d), lambda i, *_: (i, 0))
    grid_spec = pltpu.PrefetchScalarGridSpec(
        num_scalar_prefetch=2,
        grid=(NB,),
        in_specs=[
            rowblk(64), rowblk(64), rowblk(16), rowblk(4),
            full((NRP, OUT_F)), full((NRP, OUT_F)),
            full((IN_F, OUT_F)), full((IN_F, OUT_F)),
            full((IN_F, OUT_F)), full((IN_F, OUT_F)),
            full((IN_F + 2, OUT_F)), full((IN_F + 2, OUT_F)),
            full((IN_F + 2, OUT_F)), full((IN_F + 2, OUT_F)),
        ],
        out_specs=pl.BlockSpec((1, 2 * OUT_F, 2 * OUT_F),
                               lambda i, rob_ref, fob_ref: (rob_ref[i], 0, 0)),
    )
    return pl.pallas_call(
        body,
        grid_spec=grid_spec,
        out_shape=jax.ShapeDtypeStruct((NRP, 2 * OUT_F, 2 * OUT_F),
                                       jnp.float32),
    )(rob, fob, *arrays)


def _att_kernel(g):
    def body(g_ref, mr, mi):
        gg = g_ref[...]
        mr[...] = jax.nn.sigmoid(gg[:, 0:OUT_F, 0:OUT_F]
                                 - gg[:, OUT_F:2 * OUT_F, OUT_F:2 * OUT_F])
        mi[...] = jax.nn.sigmoid(gg[:, 0:OUT_F, OUT_F:2 * OUT_F]
                                 + gg[:, OUT_F:2 * OUT_F, 0:OUT_F])

    return pl.pallas_call(
        body,
        out_shape=[jax.ShapeDtypeStruct((NRP, OUT_F, OUT_F), jnp.float32)] * 2,
    )(g)


def _pass2(scalars, arrays):
    rob, fob = scalars
    NB = _PC["NB"]
    P = _PC["P"]

    def body(rob_ref, fob_ref, are, aim, m, st, sre, sim,
             w1r, w1i, w2r, w2i, mr_ref, mi_ref, fre, fim):
        i = pl.program_id(0)
        rid = rob_ref[i]
        o_re, o_im = _compute_o(are[...], aim[...], m[...], st[...],
                                sre, sim, w1r[...], w1i[...],
                                w2r[...], w2i[...], rid)
        mul_re = mr_ref[0]
        mul_im = mi_ref[0]
        fre[...] = _dot(o_re, mul_re) - _dot(o_im, mul_im)
        fim[...] = _dot(o_re, mul_im) + _dot(o_im, mul_re)

    full = lambda shape: pl.BlockSpec(shape, lambda i, *_: (0,) * len(shape))
    rowblk = lambda d: pl.BlockSpec((BLK, d), lambda i, *_: (i, 0))
    grid_spec = pltpu.PrefetchScalarGridSpec(
        num_scalar_prefetch=2,
        grid=(NB,),
        in_specs=[
            rowblk(64), rowblk(64), rowblk(16), rowblk(4),
            full((NRP, OUT_F)), full((NRP, OUT_F)),
            full((IN_F, OUT_F)), full((IN_F, OUT_F)),
            full((IN_F, OUT_F)), full((IN_F, OUT_F)),
            pl.BlockSpec((1, OUT_F, OUT_F),
                         lambda i, rob_ref, fob_ref: (rob_ref[i], 0, 0)),
            pl.BlockSpec((1, OUT_F, OUT_F),
                         lambda i, rob_ref, fob_ref: (rob_ref[i], 0, 0)),
        ],
        out_specs=[
            pl.BlockSpec((BLK, OUT_F), lambda i, *_: (i, 0)),
            pl.BlockSpec((BLK, OUT_F), lambda i, *_: (i, 0)),
        ],
    )
    return pl.pallas_call(
        body,
        grid_spec=grid_spec,
        out_shape=[jax.ShapeDtypeStruct((P, OUT_F), jnp.float32)] * 2,
    )(rob, fob, *arrays)


def kernel(input_image_re, input_image_im, input_mask, output_re, output_im,
           w1_re, w1_im, w2_re, w2_im, L1_re, L1_im, L2_re, L2_im):
    pc = _PC
    P, M = pc["P"], pc["M"]
    a_re = input_image_re.reshape(N, IN_F)
    a_im = input_image_im.reshape(N, IN_F)
    mask_t = jnp.tile(input_mask.reshape(N, 1).astype(jnp.float32), (1, 16))
    idx_all = jnp.asarray(pc["idx_all"])
    g_re, g_im, g_m = _sc_gather([a_re, a_im, mask_t], idx_all, M)

    nbr_re = lax.slice(g_re, (P, 0), (P + NR * MAX_NEIGH, IN_F))
    nbr_im = lax.slice(g_im, (P, 0), (P + NR * MAX_NEIGH, IN_F))
    nbr_m = lax.slice(g_m, (P, 0), (P + NR * MAX_NEIGH, 16))
    wfull = jnp.asarray(pc["nbr_w"])
    kcol = jnp.asarray(pc["nbr_k"])
    s_re, s_im = _s_kernel(nbr_re, nbr_im, nbr_m, wfull, kcol, w1_re, w1_im)

    rob = jnp.asarray(pc["rob"])
    fob = jnp.asarray(pc["fob"])
    st = jnp.asarray(pc["st"])
    g = _pass1((rob, fob),
               [g_re, g_im, g_m, st, s_re, s_im,
                w1_re, w1_im, w2_re, w2_im, L1_re, L1_im, L2_re, L2_im])
    mul_r, mul_i = _att_kernel(g)
    f_re, f_im = _pass2((rob, fob),
                        [g_re, g_im, g_m, st, s_re, s_im,
                         w1_re, w1_im, w2_re, w2_im, mul_r, mul_i])

    inv = jnp.asarray(pc["inv"])
    out_re, out_im = _sc_gather([f_re, f_im], inv, N)
    out_re = out_re.reshape(H, W, OUT_F)
    out_im = out_im.reshape(H, W, OUT_F)
    return out_re + 1j * out_im
